# baseline (speedup 1.0000x reference)
import functools

import jax
import jax.numpy as jnp
from jax import lax
from jax.experimental import pallas as pl
from jax.experimental.pallas import tpu as pltpu
from jax.experimental.pallas import tpu_sc as plsc

EXER_N = 10000
N_EX = 16384
KN = 64
NC = 2
NS = 16
LANES = 16

ROWS_PER_TILE = 640
CHUNKS_PER_TILE = ROWS_PER_TILE // LANES
OUT_PER_TILE = N_EX // (NC * NS)
IDX_MINOR = 128
IDX_ROWS = OUT_PER_TILE // IDX_MINOR


def _sc_body(stu_hbm, ex_hbm, q_hbm, d_hbm, u_hbm, a_hbm, g_hbm,
             state_out, pred_out,
             qrows_v, acol_v, gcol_v, dcol_v, urow_v, sidx_v, pred_v,
             exidx_v, psel_v, shared, sem):
    c = lax.axis_index("c")
    s = lax.axis_index("s")
    base_t = jnp.minimum(s * ROWS_PER_TILE, EXER_N - ROWS_PER_TILE)

    pltpu.sync_copy(stu_hbm, sidx_v)
    pltpu.async_copy(u_hbm.at[sidx_v], urow_v, sem).wait()
    pltpu.sync_copy(q_hbm.at[pl.ds(base_t * KN, ROWS_PER_TILE * KN)], qrows_v)
    pltpu.sync_copy(a_hbm.at[pl.ds(base_t, ROWS_PER_TILE)], acol_v)
    pltpu.sync_copy(g_hbm.at[pl.ds(base_t, ROWS_PER_TILE)], gcol_v)
    pltpu.sync_copy(d_hbm.at[pl.ds(base_t, ROWS_PER_TILE)], dcol_v)

    u_vecs = [urow_v[0, pl.ds(i * LANES, LANES)] for i in range(KN // LANES)]

    def chunk(cc, carry):
        row_idx = lax.iota(jnp.int32, LANES) + cc * LANES
        qbase = row_idx * KN
        acc = jnp.zeros((LANES,), jnp.float32)
        for k in range(KN):
            qv = plsc.load_gather(qrows_v, [qbase + k])
            acc = acc + qv * u_vecs[k // LANES][k % LANES]
        av = plsc.load_gather(acol_v, [row_idx])
        gv = plsc.load_gather(gcol_v, [row_idx])
        dv = plsc.load_gather(dcol_v, [row_idx])
        x = av * (acc / dv - 0.5) + gv
        pred = 1.0 / (1.0 + jnp.exp(x))
        plsc.store_scatter(pred_v, [row_idx], pred)
        return carry

    lax.fori_loop(0, CHUNKS_PER_TILE, chunk, 0)

    pltpu.sync_copy(pred_v, shared.at[pl.ds(base_t, ROWS_PER_TILE)])
    plsc.subcore_barrier()

    wid = s * NC + c
    pltpu.sync_copy(ex_hbm.at[wid], exidx_v)
    for j in range(IDX_ROWS):
        pltpu.async_copy(shared.at[exidx_v.at[j]],
                         psel_v.at[pl.ds(j * IDX_MINOR, IDX_MINOR)],
                         sem).wait()
    pltpu.sync_copy(psel_v, pred_out.at[pl.ds(wid * OUT_PER_TILE, OUT_PER_TILE)])

    @pl.when((c == 0) & (s == 0))
    def _():
        pltpu.sync_copy(urow_v.at[0], state_out)


@jax.jit
def _sc_call(stu_id, ex3, q_kn, d_flat, U_embeding, a_flat, g_flat):
    mesh = plsc.VectorSubcoreMesh(core_axis_name="c", subcore_axis_name="s")
    run = functools.partial(
        pl.kernel,
        mesh=mesh,
        compiler_params=pltpu.CompilerParams(
            needs_layout_passes=False, use_tc_tiling_on_sc=False),
        out_type=(
            jax.ShapeDtypeStruct((KN,), jnp.float32),
            jax.ShapeDtypeStruct((N_EX,), jnp.float32),
        ),
        scratch_types=[
            pltpu.VMEM((ROWS_PER_TILE * KN,), jnp.float32),
            pltpu.VMEM((ROWS_PER_TILE,), jnp.float32),
            pltpu.VMEM((ROWS_PER_TILE,), jnp.float32),
            pltpu.VMEM((ROWS_PER_TILE,), jnp.float32),
            pltpu.VMEM((1, KN), jnp.float32),
            pltpu.VMEM((1,), jnp.int32),
            pltpu.VMEM((ROWS_PER_TILE,), jnp.float32),
            pltpu.VMEM((IDX_ROWS, IDX_MINOR), jnp.int32),
            pltpu.VMEM((OUT_PER_TILE,), jnp.float32),
            pltpu.VMEM_SHARED((NS * ROWS_PER_TILE,), jnp.float32),
            pltpu.SemaphoreType.DMA,
        ],
    )(_sc_body)
    return run(stu_id, ex3, q_kn.reshape(-1), d_flat, U_embeding, a_flat,
               g_flat)


def kernel(stu_id, kn_id, score, user_k_kc, ex_id, q_kn, d, U_embeding,
           alpha, gamma_e):
    ex3 = ex_id.astype(jnp.int32).reshape(NC * NS, IDX_ROWS, IDX_MINOR)
    state_flat, pred_flat = _sc_call(
        stu_id.astype(jnp.int32), ex3, q_kn, d.reshape(-1), U_embeding,
        alpha.reshape(-1), gamma_e.reshape(-1))
    return state_flat.reshape(KN, 1), pred_flat.reshape(N_EX, 1)


# speedup vs baseline: 1.5223x; 1.5223x over previous
import functools

import jax
import jax.numpy as jnp
from jax import lax
from jax.experimental import pallas as pl
from jax.experimental.pallas import tpu as pltpu
from jax.experimental.pallas import tpu_sc as plsc

EXER_N = 10000
N_EX = 16384
KN = 64
NC = 2
NS = 16
LANES = 16

ROWS_PER_TILE = 640
CHUNKS_PER_TILE = ROWS_PER_TILE // LANES
OUT_PER_TILE = N_EX // (NC * NS)
IDX_MINOR = 128
IDX_ROWS = OUT_PER_TILE // IDX_MINOR
NACC = 4


def _row_body(stu_ref, u_ref, o_ref):
    r = stu_ref[0] % 8
    o_ref[...] = jnp.broadcast_to(u_ref[pl.ds(r, 1), :], (8, KN))


@jax.jit
def _row_call(stu_id, U_embeding):
    grid_spec = pltpu.PrefetchScalarGridSpec(
        num_scalar_prefetch=1,
        grid=(1,),
        in_specs=[pl.BlockSpec((8, KN), lambda i, stu: (stu[0] // 8, 0))],
        out_specs=pl.BlockSpec((8, KN), lambda i, stu: (0, 0)),
    )
    return pl.pallas_call(
        _row_body,
        grid_spec=grid_spec,
        out_shape=jax.ShapeDtypeStruct((8, KN), jnp.float32),
    )(stu_id, U_embeding)


def _sc_body(ex_hbm, q_hbm, d_hbm, a_hbm, g_hbm, urow_hbm,
             pred_out,
             qrows_v, acol_v, gcol_v, dcol_v, urow_v, pred_v,
             exidx_v, psel_v, shared, sem):
    c = lax.axis_index("c")
    s = lax.axis_index("s")
    base_t = jnp.minimum(s * ROWS_PER_TILE, EXER_N - ROWS_PER_TILE)

    pltpu.sync_copy(urow_hbm, urow_v)
    pltpu.sync_copy(q_hbm.at[pl.ds(base_t * KN, ROWS_PER_TILE * KN)], qrows_v)
    pltpu.sync_copy(a_hbm.at[pl.ds(base_t, ROWS_PER_TILE)], acol_v)
    pltpu.sync_copy(g_hbm.at[pl.ds(base_t, ROWS_PER_TILE)], gcol_v)
    pltpu.sync_copy(d_hbm.at[pl.ds(base_t, ROWS_PER_TILE)], dcol_v)

    u_vecs = [urow_v[0, pl.ds(i * LANES, LANES)] for i in range(KN // LANES)]

    @plsc.parallel_loop(0, CHUNKS_PER_TILE, unroll=2)
    def chunk(cc):
        row_idx = lax.iota(jnp.int32, LANES) + cc * LANES
        qbase = row_idx * KN
        accs = [jnp.zeros((LANES,), jnp.float32) for _ in range(NACC)]
        for k in range(KN):
            qv = plsc.load_gather(qrows_v, [qbase + k])
            accs[k % NACC] = accs[k % NACC] + qv * u_vecs[k // LANES][k % LANES]
        acc = (accs[0] + accs[1]) + (accs[2] + accs[3])
        av = plsc.load_gather(acol_v, [row_idx])
        gv = plsc.load_gather(gcol_v, [row_idx])
        dv = plsc.load_gather(dcol_v, [row_idx])
        x = av * (acc / dv - 0.5) + gv
        pred = 1.0 / (1.0 + jnp.exp(x))
        plsc.store_scatter(pred_v, [row_idx], pred)

    pltpu.sync_copy(pred_v, shared.at[pl.ds(base_t, ROWS_PER_TILE)])
    plsc.subcore_barrier()

    wid = s * NC + c
    pltpu.sync_copy(ex_hbm.at[pl.ds(wid * OUT_PER_TILE, OUT_PER_TILE)], exidx_v)
    for j in range(IDX_ROWS):
        pltpu.async_copy(shared.at[exidx_v.at[pl.ds(j * IDX_MINOR, IDX_MINOR)]],
                         psel_v.at[pl.ds(j * IDX_MINOR, IDX_MINOR)],
                         sem).wait()
    pltpu.sync_copy(psel_v, pred_out.at[pl.ds(wid * OUT_PER_TILE, OUT_PER_TILE)])


@jax.jit
def _sc_call(ex_id, q_flat, d_flat, a_flat, g_flat, urow):
    mesh = plsc.VectorSubcoreMesh(core_axis_name="c", subcore_axis_name="s")
    run = functools.partial(
        pl.kernel,
        mesh=mesh,
        compiler_params=pltpu.CompilerParams(
            needs_layout_passes=False, use_tc_tiling_on_sc=False),
        out_type=jax.ShapeDtypeStruct((N_EX,), jnp.float32),
        scratch_types=[
            pltpu.VMEM((ROWS_PER_TILE * KN,), jnp.float32),
            pltpu.VMEM((ROWS_PER_TILE,), jnp.float32),
            pltpu.VMEM((ROWS_PER_TILE,), jnp.float32),
            pltpu.VMEM((ROWS_PER_TILE,), jnp.float32),
            pltpu.VMEM((8, KN), jnp.float32),
            pltpu.VMEM((ROWS_PER_TILE,), jnp.float32),
            pltpu.VMEM((OUT_PER_TILE,), jnp.int32),
            pltpu.VMEM((OUT_PER_TILE,), jnp.float32),
            pltpu.VMEM_SHARED((NS * ROWS_PER_TILE,), jnp.float32),
            pltpu.SemaphoreType.DMA,
        ],
    )(_sc_body)
    return run(ex_id, q_flat, d_flat, a_flat, g_flat, urow)


def kernel(stu_id, kn_id, score, user_k_kc, ex_id, q_kn, d, U_embeding,
           alpha, gamma_e):
    urow8 = _row_call(stu_id.astype(jnp.int32), U_embeding)
    pred_flat = _sc_call(
        ex_id.astype(jnp.int32), q_kn.reshape(-1), d.reshape(-1),
        alpha.reshape(-1), gamma_e.reshape(-1), urow8)
    return urow8[0].reshape(KN, 1), pred_flat.reshape(N_EX, 1)


# speedup vs baseline: 6.0455x; 3.9713x over previous
import functools

import jax
import jax.numpy as jnp
from jax import lax
from jax.experimental import pallas as pl
from jax.experimental.pallas import tpu as pltpu
from jax.experimental.pallas import tpu_sc as plsc

EXER_N = 10000
N_EX = 16384
KN = 64
NC = 2
NS = 16
LANES = 16

ROWS_PER_TILE = 640
CHUNKS_PER_TILE = ROWS_PER_TILE // LANES
OUT_PER_TILE = N_EX // (NC * NS)
IDX_MINOR = 128
IDX_ROWS = OUT_PER_TILE // IDX_MINOR
NACC = 4


def _row_body(stu_ref, ut_ref, o_ref):
    col = stu_ref[0] % 128
    lane = lax.broadcasted_iota(jnp.int32, (KN, 128), 1)
    sel = jnp.where(lane == col, ut_ref[...], 0.0)
    o_ref[...] = jnp.sum(sel, axis=1, keepdims=True)


@jax.jit
def _row_call(stu_id, U_T):
    grid_spec = pltpu.PrefetchScalarGridSpec(
        num_scalar_prefetch=1,
        grid=(1,),
        in_specs=[pl.BlockSpec((KN, 128), lambda i, stu: (0, stu[0] // 128))],
        out_specs=pl.BlockSpec((KN, 1), lambda i, stu: (0, 0)),
    )
    return pl.pallas_call(
        _row_body,
        grid_spec=grid_spec,
        out_shape=jax.ShapeDtypeStruct((KN, 1), jnp.float32),
    )(stu_id, U_T)


def _sc_body(ex_hbm, q_hbm, d_hbm, a_hbm, g_hbm, urow_hbm,
             pred_out,
             qrows_v, acol_v, gcol_v, dcol_v, urow_v, pred_v,
             exidx_v, psel_v, shared, sem):
    c = lax.axis_index("c")
    s = lax.axis_index("s")
    base_t = jnp.minimum(s * ROWS_PER_TILE, EXER_N - ROWS_PER_TILE)

    pltpu.sync_copy(urow_hbm, urow_v)
    pltpu.sync_copy(q_hbm.at[pl.ds(base_t * KN, ROWS_PER_TILE * KN)], qrows_v)
    pltpu.sync_copy(a_hbm.at[pl.ds(base_t, ROWS_PER_TILE)], acol_v)
    pltpu.sync_copy(g_hbm.at[pl.ds(base_t, ROWS_PER_TILE)], gcol_v)
    pltpu.sync_copy(d_hbm.at[pl.ds(base_t, ROWS_PER_TILE)], dcol_v)

    u_vecs = [urow_v[pl.ds(i * LANES, LANES)] for i in range(KN // LANES)]

    @plsc.parallel_loop(0, CHUNKS_PER_TILE, unroll=2)
    def chunk(cc):
        row_idx = lax.iota(jnp.int32, LANES) + cc * LANES
        qbase = row_idx * KN
        accs = [jnp.zeros((LANES,), jnp.float32) for _ in range(NACC)]
        for k in range(KN):
            qv = plsc.load_gather(qrows_v, [qbase + k])
            accs[k % NACC] = accs[k % NACC] + qv * u_vecs[k // LANES][k % LANES]
        acc = (accs[0] + accs[1]) + (accs[2] + accs[3])
        av = plsc.load_gather(acol_v, [row_idx])
        gv = plsc.load_gather(gcol_v, [row_idx])
        dv = plsc.load_gather(dcol_v, [row_idx])
        x = av * (acc / dv - 0.5) + gv
        pred = 1.0 / (1.0 + jnp.exp(x))
        plsc.store_scatter(pred_v, [row_idx], pred)

    pltpu.sync_copy(pred_v, shared.at[pl.ds(base_t, ROWS_PER_TILE)])
    plsc.subcore_barrier()

    wid = s * NC + c
    pltpu.sync_copy(ex_hbm.at[pl.ds(wid * OUT_PER_TILE, OUT_PER_TILE)], exidx_v)
    for j in range(IDX_ROWS):
        pltpu.async_copy(shared.at[exidx_v.at[pl.ds(j * IDX_MINOR, IDX_MINOR)]],
                         psel_v.at[pl.ds(j * IDX_MINOR, IDX_MINOR)],
                         sem).wait()
    pltpu.sync_copy(psel_v, pred_out.at[pl.ds(wid * OUT_PER_TILE, OUT_PER_TILE)])


@jax.jit
def _sc_call(ex_id, q_flat, d_flat, a_flat, g_flat, urow):
    mesh = plsc.VectorSubcoreMesh(core_axis_name="c", subcore_axis_name="s")
    run = functools.partial(
        pl.kernel,
        mesh=mesh,
        compiler_params=pltpu.CompilerParams(
            needs_layout_passes=False, use_tc_tiling_on_sc=False),
        out_type=jax.ShapeDtypeStruct((N_EX,), jnp.float32),
        scratch_types=[
            pltpu.VMEM((ROWS_PER_TILE * KN,), jnp.float32),
            pltpu.VMEM((ROWS_PER_TILE,), jnp.float32),
            pltpu.VMEM((ROWS_PER_TILE,), jnp.float32),
            pltpu.VMEM((ROWS_PER_TILE,), jnp.float32),
            pltpu.VMEM((KN,), jnp.float32),
            pltpu.VMEM((ROWS_PER_TILE,), jnp.float32),
            pltpu.VMEM((OUT_PER_TILE,), jnp.int32),
            pltpu.VMEM((OUT_PER_TILE,), jnp.float32),
            pltpu.VMEM_SHARED((NS * ROWS_PER_TILE,), jnp.float32),
            pltpu.SemaphoreType.DMA,
        ],
    )(_sc_body)
    return run(ex_id, q_flat, d_flat, a_flat, g_flat, urow)


def kernel(stu_id, kn_id, score, user_k_kc, ex_id, q_kn, d, U_embeding,
           alpha, gamma_e):
    ucol = _row_call(stu_id.astype(jnp.int32), U_embeding.T)
    pred_flat = _sc_call(
        ex_id.astype(jnp.int32), q_kn.reshape(-1), d.reshape(-1),
        alpha.reshape(-1), gamma_e.reshape(-1), ucol.reshape(KN))
    return ucol, pred_flat.reshape(N_EX, 1)


# speedup vs baseline: 9.4847x; 1.5689x over previous
import functools

import jax
import jax.numpy as jnp
from jax import lax
from jax.experimental import pallas as pl
from jax.experimental.pallas import tpu as pltpu
from jax.experimental.pallas import tpu_sc as plsc

EXER_N = 10000
N_EX = 16384
KN = 64
NC = 2
NS = 16
LANES = 16

ROWS_PER_TILE = 640
CHUNKS_PER_TILE = ROWS_PER_TILE // LANES
OUT_PER_TILE = N_EX // (NC * NS)
IDX_MINOR = 128
IDX_ROWS = OUT_PER_TILE // IDX_MINOR
NACC = 4


def _row_body(stu_ref, ut_ref, o_ref):
    col = stu_ref[0] % 128
    lane = lax.broadcasted_iota(jnp.int32, (KN, 128), 1)
    sel = jnp.where(lane == col, ut_ref[...], 0.0)
    o_ref[...] = jnp.sum(sel, axis=1, keepdims=True)


@jax.jit
def _row_call(stu_id, U_T):
    grid_spec = pltpu.PrefetchScalarGridSpec(
        num_scalar_prefetch=1,
        grid=(1,),
        in_specs=[pl.BlockSpec((KN, 128), lambda i, stu: (0, stu[0] // 128))],
        out_specs=pl.BlockSpec((KN, 1), lambda i, stu: (0, 0)),
    )
    return pl.pallas_call(
        _row_body,
        grid_spec=grid_spec,
        out_shape=jax.ShapeDtypeStruct((KN, 1), jnp.float32),
    )(stu_id, U_T)


def _sc_body(ex_hbm, q_hbm, d_hbm, a_hbm, g_hbm, urow_hbm,
             pred_out,
             qrows_v, acol_v, gcol_v, dcol_v, urow_v, pred_v,
             exidx_v, psel_v, shared, sem):
    c = lax.axis_index("c")
    s = lax.axis_index("s")
    base_t = jnp.minimum(s * ROWS_PER_TILE, EXER_N - ROWS_PER_TILE)

    pltpu.sync_copy(urow_hbm, urow_v)
    for k in range(KN):
        pltpu.async_copy(
            q_hbm.at[pl.ds(k * EXER_N + base_t, ROWS_PER_TILE)],
            qrows_v.at[pl.ds(k * ROWS_PER_TILE, ROWS_PER_TILE)], sem)
    pltpu.sync_copy(a_hbm.at[pl.ds(base_t, ROWS_PER_TILE)], acol_v)
    pltpu.sync_copy(g_hbm.at[pl.ds(base_t, ROWS_PER_TILE)], gcol_v)
    pltpu.sync_copy(d_hbm.at[pl.ds(base_t, ROWS_PER_TILE)], dcol_v)
    for k in range(KN):
        pltpu.make_async_copy(
            q_hbm.at[pl.ds(k * EXER_N + base_t, ROWS_PER_TILE)],
            qrows_v.at[pl.ds(k * ROWS_PER_TILE, ROWS_PER_TILE)], sem).wait()

    u_vecs = [urow_v[pl.ds(i * LANES, LANES)] for i in range(KN // LANES)]

    @plsc.parallel_loop(0, CHUNKS_PER_TILE, unroll=2)
    def chunk(cc):
        col = cc * LANES
        accs = [jnp.zeros((LANES,), jnp.float32) for _ in range(NACC)]
        for k in range(KN):
            qv = qrows_v[pl.ds(k * ROWS_PER_TILE + col, LANES)]
            accs[k % NACC] = accs[k % NACC] + qv * u_vecs[k // LANES][k % LANES]
        acc = (accs[0] + accs[1]) + (accs[2] + accs[3])
        av = acol_v[pl.ds(col, LANES)]
        gv = gcol_v[pl.ds(col, LANES)]
        dv = dcol_v[pl.ds(col, LANES)]
        x = av * (acc / dv - 0.5) + gv
        pred = 1.0 / (1.0 + jnp.exp(x))
        pred_v[pl.ds(col, LANES)] = pred

    pltpu.sync_copy(pred_v, shared.at[pl.ds(base_t, ROWS_PER_TILE)])
    plsc.subcore_barrier()

    wid = s * NC + c
    pltpu.sync_copy(ex_hbm.at[pl.ds(wid * OUT_PER_TILE, OUT_PER_TILE)], exidx_v)
    for j in range(IDX_ROWS):
        pltpu.async_copy(shared.at[exidx_v.at[pl.ds(j * IDX_MINOR, IDX_MINOR)]],
                         psel_v.at[pl.ds(j * IDX_MINOR, IDX_MINOR)],
                         sem).wait()
    pltpu.sync_copy(psel_v, pred_out.at[pl.ds(wid * OUT_PER_TILE, OUT_PER_TILE)])


@jax.jit
def _sc_call(ex_id, q_flat, d_flat, a_flat, g_flat, urow):
    mesh = plsc.VectorSubcoreMesh(core_axis_name="c", subcore_axis_name="s")
    run = functools.partial(
        pl.kernel,
        mesh=mesh,
        compiler_params=pltpu.CompilerParams(
            needs_layout_passes=False, use_tc_tiling_on_sc=False),
        out_type=jax.ShapeDtypeStruct((N_EX,), jnp.float32),
        scratch_types=[
            pltpu.VMEM((ROWS_PER_TILE * KN,), jnp.float32),
            pltpu.VMEM((ROWS_PER_TILE,), jnp.float32),
            pltpu.VMEM((ROWS_PER_TILE,), jnp.float32),
            pltpu.VMEM((ROWS_PER_TILE,), jnp.float32),
            pltpu.VMEM((KN,), jnp.float32),
            pltpu.VMEM((ROWS_PER_TILE,), jnp.float32),
            pltpu.VMEM((OUT_PER_TILE,), jnp.int32),
            pltpu.VMEM((OUT_PER_TILE,), jnp.float32),
            pltpu.VMEM_SHARED((NS * ROWS_PER_TILE,), jnp.float32),
            pltpu.SemaphoreType.DMA,
        ],
    )(_sc_body)
    return run(ex_id, q_flat, d_flat, a_flat, g_flat, urow)


def kernel(stu_id, kn_id, score, user_k_kc, ex_id, q_kn, d, U_embeding,
           alpha, gamma_e):
    ucol = _row_call(stu_id.astype(jnp.int32), U_embeding.T)
    pred_flat = _sc_call(
        ex_id.astype(jnp.int32), q_kn.T.reshape(-1), d.reshape(-1),
        alpha.reshape(-1), gamma_e.reshape(-1), ucol.reshape(KN))
    return ucol, pred_flat.reshape(N_EX, 1)


# speedup vs baseline: 13.4226x; 1.4152x over previous
import functools

import jax
import jax.numpy as jnp
from jax import lax
from jax.experimental import pallas as pl
from jax.experimental.pallas import tpu as pltpu
from jax.experimental.pallas import tpu_sc as plsc

EXER_N = 10000
N_EX = 16384
KN = 64
NC = 2
NS = 16
LANES = 16

OUT_PER_TILE = N_EX // (NC * NS)
GATHER_CHUNKS = OUT_PER_TILE // LANES


def _dense_body(stu_ref, ut_ref, qt_ref, dt_ref, at_ref, gt_ref,
                pred_ref, state_ref):
    col = stu_ref[0] % 128
    lane = lax.broadcasted_iota(jnp.int32, (KN, 128), 1)
    u = jnp.sum(jnp.where(lane == col, ut_ref[...], 0.0), axis=1,
                keepdims=True)
    state_ref[...] = u
    v = jnp.sum(qt_ref[...] * u, axis=0, keepdims=True)
    x = at_ref[...] * (v / dt_ref[...] - 0.5) + gt_ref[...]
    pred_ref[...] = 1.0 / (1.0 + jnp.exp(x))


@jax.jit
def _dense_call(stu_id, U_T, q_T, d_T, a_T, g_T):
    grid_spec = pltpu.PrefetchScalarGridSpec(
        num_scalar_prefetch=1,
        grid=(1,),
        in_specs=[
            pl.BlockSpec((KN, 128), lambda i, stu: (0, stu[0] // 128)),
            pl.BlockSpec((KN, EXER_N), lambda i, stu: (0, 0)),
            pl.BlockSpec((1, EXER_N), lambda i, stu: (0, 0)),
            pl.BlockSpec((1, EXER_N), lambda i, stu: (0, 0)),
            pl.BlockSpec((1, EXER_N), lambda i, stu: (0, 0)),
        ],
        out_specs=[
            pl.BlockSpec((1, EXER_N), lambda i, stu: (0, 0)),
            pl.BlockSpec((KN, 1), lambda i, stu: (0, 0)),
        ],
    )
    return pl.pallas_call(
        _dense_body,
        grid_spec=grid_spec,
        out_shape=(
            jax.ShapeDtypeStruct((1, EXER_N), jnp.float32),
            jax.ShapeDtypeStruct((KN, 1), jnp.float32),
        ),
    )(stu_id, U_T, q_T, d_T, a_T, g_T)


def _sc_body(ex_hbm, tab_hbm, pred_out, tab_v, exidx_v, psel_v, sem):
    c = lax.axis_index("c")
    s = lax.axis_index("s")
    wid = s * NC + c
    pltpu.sync_copy(ex_hbm.at[pl.ds(wid * OUT_PER_TILE, OUT_PER_TILE)],
                    exidx_v)
    pltpu.sync_copy(tab_hbm, tab_v)

    @plsc.parallel_loop(0, GATHER_CHUNKS, unroll=4)
    def chunk(cc):
        col = cc * LANES
        idx = exidx_v[pl.ds(col, LANES)]
        psel_v[pl.ds(col, LANES)] = plsc.load_gather(tab_v, [idx])

    pltpu.sync_copy(psel_v, pred_out.at[pl.ds(wid * OUT_PER_TILE,
                                              OUT_PER_TILE)])


@jax.jit
def _sc_call(ex_id, table):
    mesh = plsc.VectorSubcoreMesh(core_axis_name="c", subcore_axis_name="s")
    run = functools.partial(
        pl.kernel,
        mesh=mesh,
        compiler_params=pltpu.CompilerParams(
            needs_layout_passes=False, use_tc_tiling_on_sc=False),
        out_type=jax.ShapeDtypeStruct((N_EX,), jnp.float32),
        scratch_types=[
            pltpu.VMEM((EXER_N,), jnp.float32),
            pltpu.VMEM((OUT_PER_TILE,), jnp.int32),
            pltpu.VMEM((OUT_PER_TILE,), jnp.float32),
            pltpu.SemaphoreType.DMA,
        ],
    )(_sc_body)
    return run(ex_id, table)


def kernel(stu_id, kn_id, score, user_k_kc, ex_id, q_kn, d, U_embeding,
           alpha, gamma_e):
    table, state = _dense_call(
        stu_id.astype(jnp.int32), U_embeding.T, q_kn.T, d.T, alpha.T,
        gamma_e.T)
    pred_flat = _sc_call(ex_id.astype(jnp.int32), table.reshape(EXER_N))
    return state, pred_flat.reshape(N_EX, 1)


# speedup vs baseline: 14.4740x; 1.0783x over previous
import functools

import jax
import jax.numpy as jnp
from jax import lax
from jax.experimental import pallas as pl
from jax.experimental.pallas import tpu as pltpu
from jax.experimental.pallas import tpu_sc as plsc

EXER_N = 10000
N_EX = 16384
KN = 64
NC = 2
NS = 16
LANES = 16

OUT_PER_TILE = N_EX // (NC * NS)
GATHER_CHUNKS = OUT_PER_TILE // LANES


def _dense_body(stu_ref, ut_ref, qt_ref, dt_ref, at_ref, gt_ref,
                pred_ref, state_ref):
    col = stu_ref[0] % 128
    lane = lax.broadcasted_iota(jnp.int32, (KN, 128), 1)
    u = jnp.sum(jnp.where(lane == col, ut_ref[...], 0.0), axis=1,
                keepdims=True)
    state_ref[...] = u
    v = jnp.sum(qt_ref[...] * u, axis=0, keepdims=True)
    x = at_ref[...] * (v / dt_ref[...] - 0.5) + gt_ref[...]
    pred_ref[...] = (1.0 / (1.0 + jnp.exp(x))).reshape(EXER_N)


@jax.jit
def _dense_call(stu_id, U_T, q_T, d_T, a_T, g_T):
    grid_spec = pltpu.PrefetchScalarGridSpec(
        num_scalar_prefetch=1,
        grid=(1,),
        in_specs=[
            pl.BlockSpec((KN, 128), lambda i, stu: (0, stu[0] // 128)),
            pl.BlockSpec((KN, EXER_N), lambda i, stu: (0, 0)),
            pl.BlockSpec((1, EXER_N), lambda i, stu: (0, 0)),
            pl.BlockSpec((1, EXER_N), lambda i, stu: (0, 0)),
            pl.BlockSpec((1, EXER_N), lambda i, stu: (0, 0)),
        ],
        out_specs=[
            pl.BlockSpec((EXER_N,), lambda i, stu: (0,)),
            pl.BlockSpec((KN, 1), lambda i, stu: (0, 0)),
        ],
    )
    return pl.pallas_call(
        _dense_body,
        grid_spec=grid_spec,
        out_shape=(
            jax.ShapeDtypeStruct((EXER_N,), jnp.float32),
            jax.ShapeDtypeStruct((KN, 1), jnp.float32),
        ),
    )(stu_id, U_T, q_T, d_T, a_T, g_T)


def _sc_body(ex_hbm, tab_hbm, pred_out, tab_v, exidx_v, psel_v, sem):
    c = lax.axis_index("c")
    s = lax.axis_index("s")
    wid = s * NC + c
    pltpu.sync_copy(ex_hbm.at[pl.ds(wid * OUT_PER_TILE, OUT_PER_TILE)],
                    exidx_v)
    pltpu.sync_copy(tab_hbm, tab_v)

    @plsc.parallel_loop(0, GATHER_CHUNKS, unroll=4)
    def chunk(cc):
        col = cc * LANES
        idx = exidx_v[pl.ds(col, LANES)]
        psel_v[pl.ds(col, LANES)] = plsc.load_gather(tab_v, [idx])

    pltpu.sync_copy(psel_v, pred_out.at[pl.ds(wid * OUT_PER_TILE,
                                              OUT_PER_TILE)])


@jax.jit
def _sc_call(ex_id, table):
    mesh = plsc.VectorSubcoreMesh(core_axis_name="c", subcore_axis_name="s")
    run = functools.partial(
        pl.kernel,
        mesh=mesh,
        compiler_params=pltpu.CompilerParams(
            needs_layout_passes=False, use_tc_tiling_on_sc=False,
            skip_device_barrier=True),
        out_type=jax.ShapeDtypeStruct((N_EX,), jnp.float32),
        scratch_types=[
            pltpu.VMEM((EXER_N,), jnp.float32),
            pltpu.VMEM((OUT_PER_TILE,), jnp.int32),
            pltpu.VMEM((OUT_PER_TILE,), jnp.float32),
            pltpu.SemaphoreType.DMA,
        ],
    )(_sc_body)
    return run(ex_id, table)


def kernel(stu_id, kn_id, score, user_k_kc, ex_id, q_kn, d, U_embeding,
           alpha, gamma_e):
    table, state = _dense_call(
        stu_id.astype(jnp.int32), U_embeding.T, q_kn.T, d.T, alpha.T,
        gamma_e.T)
    pred_flat = _sc_call(ex_id.astype(jnp.int32), table)
    return state, pred_flat.reshape(N_EX, 1)


# speedup vs baseline: 14.7937x; 1.0221x over previous
import functools

import jax
import jax.numpy as jnp
from jax import lax
from jax.experimental import pallas as pl
from jax.experimental.pallas import tpu as pltpu
from jax.experimental.pallas import tpu_sc as plsc

EXER_N = 10000
N_EX = 16384
KN = 64
NC = 2
NS = 16
LANES = 16

OUT_PER_TILE = N_EX // (NC * NS)
IDX_MINOR = 128
IDX_ROWS = OUT_PER_TILE // IDX_MINOR


def _dense_body(stu_ref, ut_ref, qt_ref, dt_ref, at_ref, gt_ref,
                pred_ref, state_ref):
    col = stu_ref[0] % 128
    lane = lax.broadcasted_iota(jnp.int32, (KN, 128), 1)
    u = jnp.sum(jnp.where(lane == col, ut_ref[...], 0.0), axis=1,
                keepdims=True)
    state_ref[...] = u
    v = jnp.sum(qt_ref[...] * u, axis=0, keepdims=True)
    x = at_ref[...] * (v / dt_ref[...] - 0.5) + gt_ref[...]
    pred_ref[...] = (1.0 / (1.0 + jnp.exp(x))).reshape(EXER_N)


@jax.jit
def _dense_call(stu_id, U_T, q_T, d_T, a_T, g_T):
    grid_spec = pltpu.PrefetchScalarGridSpec(
        num_scalar_prefetch=1,
        grid=(1,),
        in_specs=[
            pl.BlockSpec((KN, 128), lambda i, stu: (0, stu[0] // 128)),
            pl.BlockSpec((KN, EXER_N), lambda i, stu: (0, 0)),
            pl.BlockSpec((1, EXER_N), lambda i, stu: (0, 0)),
            pl.BlockSpec((1, EXER_N), lambda i, stu: (0, 0)),
            pl.BlockSpec((1, EXER_N), lambda i, stu: (0, 0)),
        ],
        out_specs=[
            pl.BlockSpec((EXER_N,), lambda i, stu: (0,)),
            pl.BlockSpec((KN, 1), lambda i, stu: (0, 0)),
        ],
    )
    return pl.pallas_call(
        _dense_body,
        grid_spec=grid_spec,
        out_shape=(
            jax.ShapeDtypeStruct((EXER_N,), jnp.float32),
            jax.ShapeDtypeStruct((KN, 1), jnp.float32),
        ),
    )(stu_id, U_T, q_T, d_T, a_T, g_T)


def _sc_body(ex_hbm, tab_hbm, pred_out, exidx_v, psel_v, sem):
    c = lax.axis_index("c")
    s = lax.axis_index("s")
    wid = s * NC + c
    pltpu.sync_copy(ex_hbm.at[pl.ds(wid * OUT_PER_TILE, OUT_PER_TILE)],
                    exidx_v)
    handles = [
        pltpu.async_copy(
            tab_hbm.at[exidx_v.at[pl.ds(j * IDX_MINOR, IDX_MINOR)]],
            psel_v.at[pl.ds(j * IDX_MINOR, IDX_MINOR)], sem)
        for j in range(IDX_ROWS)
    ]
    for h in handles:
        h.wait()
    pltpu.sync_copy(psel_v, pred_out.at[pl.ds(wid * OUT_PER_TILE,
                                              OUT_PER_TILE)])


@jax.jit
def _sc_call(ex_id, table):
    mesh = plsc.VectorSubcoreMesh(core_axis_name="c", subcore_axis_name="s")
    run = functools.partial(
        pl.kernel,
        mesh=mesh,
        compiler_params=pltpu.CompilerParams(
            needs_layout_passes=False, use_tc_tiling_on_sc=False,
            skip_device_barrier=True),
        out_type=jax.ShapeDtypeStruct((N_EX,), jnp.float32),
        scratch_types=[
            pltpu.VMEM((OUT_PER_TILE,), jnp.int32),
            pltpu.VMEM((OUT_PER_TILE,), jnp.float32),
            pltpu.SemaphoreType.DMA,
        ],
    )(_sc_body)
    return run(ex_id, table)


def kernel(stu_id, kn_id, score, user_k_kc, ex_id, q_kn, d, U_embeding,
           alpha, gamma_e):
    table, state = _dense_call(
        stu_id.astype(jnp.int32), U_embeding.T, q_kn.T, d.T, alpha.T,
        gamma_e.T)
    pred_flat = _sc_call(ex_id.astype(jnp.int32), table)
    return state, pred_flat.reshape(N_EX, 1)
